# baseline (speedup 1.0000x reference)
import functools

import jax
import jax.numpy as jnp
from jax.experimental import pallas as pl
from jax.experimental.pallas import tpu as pltpu

_K = 64


def _enc_body(x_ref, w_ref, benc_ref, bdec_ref, pre_ref, m8_ref):
    xc = x_ref[...] - bdec_ref[...]
    acc = jax.lax.dot_general(
        xc, w_ref[...],
        dimension_numbers=(((1,), (1,)), ((), ())),
        preferred_element_type=jnp.float32)
    pre = acc + benc_ref[...]
    pre_ref[...] = pre
    g = pre.shape[1] // 8
    m8 = pre[:, :g]
    for j in range(1, 8):
        m8 = jnp.maximum(m8, pre[:, j * g:(j + 1) * g])
    m8_ref[...] = m8


def _encode(x, w_enc, benc2, bdec2, br, bs):
    n_tok, d_in = x.shape
    d_sae = w_enc.shape[0]
    grid = (n_tok // br, d_sae // bs)
    return pl.pallas_call(
        _enc_body,
        grid=grid,
        in_specs=[
            pl.BlockSpec((br, d_in), lambda t, s: (t, 0)),
            pl.BlockSpec((bs, d_in), lambda t, s: (s, 0)),
            pl.BlockSpec((1, bs), lambda t, s: (0, s)),
            pl.BlockSpec((1, d_in), lambda t, s: (0, 0)),
        ],
        out_specs=[
            pl.BlockSpec((br, bs), lambda t, s: (t, s)),
            pl.BlockSpec((br, bs // 8), lambda t, s: (t, s)),
        ],
        out_shape=[
            jax.ShapeDtypeStruct((n_tok, d_sae), jnp.float32),
            jax.ShapeDtypeStruct((n_tok, d_sae // 8), jnp.float32),
        ],
        compiler_params=pltpu.CompilerParams(
            dimension_semantics=("parallel", "parallel")),
    )(x, w_enc, benc2, bdec2)


def _f32_keys(v):
    kbits = jax.lax.bitcast_convert_type(v, jnp.int32)
    return jnp.where(kbits < 0, kbits ^ jnp.int32(0x7FFFFFFF), kbits)


def _bitbuild(count_fn, k, base0, b_start):
    found0 = jnp.zeros_like(base0)

    def cond(carry):
        b, base, thr, found = carry
        return (b >= 0) & (jnp.min(found) == 0)

    def body(carry):
        b, base, thr, found = carry
        cand = base + (jnp.int32(1) << b)
        cnt = count_fn(cand)
        hit = ((cnt == k) & (found == 0)).astype(jnp.int32)
        thr = jnp.where(hit == 1, cand, thr)
        found = found | hit
        base = jnp.where(cnt >= k, cand, base)
        return (b - jnp.int32(1), base, thr, found)

    _, base, thr, found = jax.lax.while_loop(
        cond, body, (b_start, base0, base0, found0))
    return jnp.where(found == 1, thr, base)


def _sel_body(pre_ref, m8_ref, thr_ref, key_ref):
    pre = pre_ref[...]
    key_ref[...] = _f32_keys(pre)
    km8 = _f32_keys(m8_ref[...])

    w = km8.shape[1]
    f = max(1, min(8, w // _K))
    g = w // f
    m64 = km8[:, :g]
    for j in range(1, f):
        m64 = jnp.maximum(m64, km8[:, j * g:(j + 1) * g])
    lb = jnp.min(m64, axis=1, keepdims=True)

    ub = jnp.max(m64, axis=1, keepdims=True)
    d = jnp.max(ub - lb)
    df = d.astype(jnp.float32)
    b_start = jnp.maximum(
        (jax.lax.bitcast_convert_type(df, jnp.int32) >> 23) - 127,
        jnp.int32(0))

    def count_full(cand):
        return jnp.sum((key_ref[...] >= cand).astype(jnp.int32),
                       axis=1, keepdims=True)

    thr_ref[...] = _bitbuild(count_full, _K, lb, b_start)


def _threshold(pre, m8, br):
    n_tok, d_sae = pre.shape
    return pl.pallas_call(
        _sel_body,
        grid=(n_tok // br,),
        in_specs=[
            pl.BlockSpec((br, d_sae), lambda t: (t, 0)),
            pl.BlockSpec((br, d_sae // 8), lambda t: (t, 0)),
        ],
        out_specs=pl.BlockSpec((br, 1), lambda t: (t, 0)),
        out_shape=jax.ShapeDtypeStruct((n_tok, 1), jnp.int32),
        scratch_shapes=[pltpu.VMEM((br, d_sae), jnp.int32)],
        compiler_params=pltpu.CompilerParams(
            dimension_semantics=("parallel",)),
    )(pre, m8)


def _dec_body(pre_ref, thr_ref, w_ref, bdec_ref, out_ref, acts_ref):
    k = pl.program_id(1)
    pre = pre_ref[...]
    keys = _f32_keys(pre)
    thr = thr_ref[...]
    acts = jnp.where((keys >= thr) & (pre > 0.0), pre, 0.0)
    acts_ref[...] = acts
    part = jax.lax.dot_general(
        acts.astype(jnp.bfloat16), w_ref[...],
        dimension_numbers=(((1,), (1,)), ((), ())),
        preferred_element_type=jnp.float32)

    @pl.when(k == 0)
    def _init():
        out_ref[...] = part + bdec_ref[...]

    @pl.when(k > 0)
    def _acc():
        out_ref[...] += part


def _decode(pre, thr, wdec_t_bf, bdec2, br, bk):
    n_tok, d_sae = pre.shape
    d_in = wdec_t_bf.shape[0]
    grid = (n_tok // br, d_sae // bk)
    return pl.pallas_call(
        _dec_body,
        grid=grid,
        in_specs=[
            pl.BlockSpec((br, bk), lambda t, k: (t, k)),
            pl.BlockSpec((br, 1), lambda t, k: (t, 0)),
            pl.BlockSpec((d_in, bk), lambda t, k: (0, k)),
            pl.BlockSpec((1, d_in), lambda t, k: (0, 0)),
        ],
        out_specs=[
            pl.BlockSpec((br, d_in), lambda t, k: (t, 0)),
            pl.BlockSpec((br, bk), lambda t, k: (t, k)),
        ],
        out_shape=[
            jax.ShapeDtypeStruct((n_tok, d_in), jnp.float32),
            jax.ShapeDtypeStruct((n_tok, d_sae), jnp.float32),
        ],
        compiler_params=pltpu.CompilerParams(
            dimension_semantics=("parallel", "arbitrary")),
    )(pre, thr, wdec_t_bf, bdec2)


def kernel(x, W_enc, b_enc, W_dec, b_dec):
    n_tok, d_in = x.shape
    d_sae = W_enc.shape[0]
    benc2 = b_enc.reshape(1, d_sae)
    bdec2 = b_dec.reshape(1, d_in)

    br1 = min(1024, n_tok)
    bs1 = min(1024, d_sae)
    pre, m8 = _encode(x, W_enc, benc2, bdec2, br1, bs1)

    br2 = min(128, n_tok)
    thr = _threshold(pre, m8, br2)

    wdec_bf = W_dec.astype(jnp.bfloat16)
    br3 = min(1024, n_tok)
    bk3 = min(1024, d_sae)
    recon, acts = _decode(pre, thr, wdec_bf, bdec2, br3, bk3)
    return (recon, acts)


# speedup vs baseline: 1.0231x; 1.0231x over previous
import functools

import jax
import jax.numpy as jnp
from jax.experimental import pallas as pl
from jax.experimental.pallas import tpu as pltpu

_K = 64


def _enc_body(x_ref, w_ref, benc_ref, bdec_ref, pre_ref, m8_ref):
    xc = x_ref[...] - bdec_ref[...]
    acc = jax.lax.dot_general(
        xc, w_ref[...],
        dimension_numbers=(((1,), (1,)), ((), ())),
        preferred_element_type=jnp.float32)
    pre = acc + benc_ref[...]
    pre_ref[...] = pre
    g = pre.shape[1] // 8
    m8 = pre[:, :g]
    for j in range(1, 8):
        m8 = jnp.maximum(m8, pre[:, j * g:(j + 1) * g])
    m8_ref[...] = m8


def _encode(x, w_enc, benc2, bdec2, br, bs):
    n_tok, d_in = x.shape
    d_sae = w_enc.shape[0]
    grid = (n_tok // br, d_sae // bs)
    return pl.pallas_call(
        _enc_body,
        grid=grid,
        in_specs=[
            pl.BlockSpec((br, d_in), lambda t, s: (t, 0)),
            pl.BlockSpec((bs, d_in), lambda t, s: (s, 0)),
            pl.BlockSpec((1, bs), lambda t, s: (0, s)),
            pl.BlockSpec((1, d_in), lambda t, s: (0, 0)),
        ],
        out_specs=[
            pl.BlockSpec((br, bs), lambda t, s: (t, s)),
            pl.BlockSpec((br, bs // 8), lambda t, s: (t, s)),
        ],
        out_shape=[
            jax.ShapeDtypeStruct((n_tok, d_sae), jnp.float32),
            jax.ShapeDtypeStruct((n_tok, d_sae // 8), jnp.float32),
        ],
        compiler_params=pltpu.CompilerParams(
            dimension_semantics=("parallel", "parallel")),
    )(x, w_enc, benc2, bdec2)


def _f32_keys(v):
    kbits = jax.lax.bitcast_convert_type(v, jnp.int32)
    return jnp.where(kbits < 0, kbits ^ jnp.int32(0x7FFFFFFF), kbits)


def _bitbuild(count_fn, k, base0, b_start):
    found0 = jnp.zeros_like(base0)

    def step(b, base, thr, found):
        cand = base + (jnp.int32(1) << b)
        cnt = count_fn(cand)
        hit = ((cnt == k) & (found == 0)).astype(jnp.int32)
        thr = jnp.where(hit == 1, cand, thr)
        found = found | hit
        base = jnp.where(cnt >= k, cand, base)
        return base, thr, found

    def cond(carry):
        b, base, thr, found = carry
        return (b >= 0) & (jnp.min(found) == 0)

    def body(carry):
        b, base, thr, found = carry
        base, thr, found = step(b, base, thr, found)
        b1 = jnp.maximum(b - 1, 0)
        base2, thr2, found2 = step(b1, base, thr, found)
        keep = b >= 1
        base = jnp.where(keep, base2, base)
        thr = jnp.where(keep, thr2, thr)
        found = jnp.where(keep, found2, found)
        return (b - jnp.int32(2), base, thr, found)

    _, base, thr, found = jax.lax.while_loop(
        cond, body, (b_start, base0, base0, found0))
    return jnp.where(found == 1, thr, base)


def _sel_body(pre_ref, m8_ref, acts_ref, key_ref):
    pre = pre_ref[...]
    key_ref[...] = _f32_keys(pre)
    km8 = _f32_keys(m8_ref[...])

    w = km8.shape[1]
    f = max(1, min(8, w // _K))
    g = w // f
    m64 = km8[:, :g]
    for j in range(1, f):
        m64 = jnp.maximum(m64, km8[:, j * g:(j + 1) * g])
    lb = jnp.min(m64, axis=1, keepdims=True)

    ub = jnp.max(m64, axis=1, keepdims=True)
    d = jnp.max(ub - lb)
    df = d.astype(jnp.float32)
    b_start = jnp.maximum(
        (jax.lax.bitcast_convert_type(df, jnp.int32) >> 23) - 127,
        jnp.int32(0))

    def count_full(cand):
        return jnp.sum((key_ref[...] >= cand).astype(jnp.int32),
                       axis=1, keepdims=True)

    thr = _bitbuild(count_full, _K, lb, b_start)
    keyv = key_ref[...]
    acts_ref[...] = jnp.where((keyv >= thr) & (pre > 0.0), pre, 0.0)


def _select(pre, m8, br):
    n_tok, d_sae = pre.shape
    return pl.pallas_call(
        _sel_body,
        grid=(n_tok // br,),
        in_specs=[
            pl.BlockSpec((br, d_sae), lambda t: (t, 0)),
            pl.BlockSpec((br, d_sae // 8), lambda t: (t, 0)),
        ],
        out_specs=pl.BlockSpec((br, d_sae), lambda t: (t, 0)),
        out_shape=jax.ShapeDtypeStruct((n_tok, d_sae), jnp.float32),
        scratch_shapes=[pltpu.VMEM((br, d_sae), jnp.int32)],
        compiler_params=pltpu.CompilerParams(
            dimension_semantics=("parallel",)),
    )(pre, m8)


def _dec_body(acts_ref, w_ref, bdec_ref, out_ref):
    k = pl.program_id(1)
    a = acts_ref[...].astype(jnp.bfloat16)
    part = jax.lax.dot_general(
        a, w_ref[...],
        dimension_numbers=(((1,), (1,)), ((), ())),
        preferred_element_type=jnp.float32)

    @pl.when(k == 0)
    def _init():
        out_ref[...] = part + bdec_ref[...]

    @pl.when(k > 0)
    def _acc():
        out_ref[...] += part


def _decode(acts, wdec_t_bf, bdec2, br, bk):
    n_tok, d_sae = acts.shape
    d_in = wdec_t_bf.shape[0]
    grid = (n_tok // br, d_sae // bk)
    return pl.pallas_call(
        _dec_body,
        grid=grid,
        in_specs=[
            pl.BlockSpec((br, bk), lambda t, k: (t, k)),
            pl.BlockSpec((d_in, bk), lambda t, k: (0, k)),
            pl.BlockSpec((1, d_in), lambda t, k: (0, 0)),
        ],
        out_specs=pl.BlockSpec((br, d_in), lambda t, k: (t, 0)),
        out_shape=jax.ShapeDtypeStruct((n_tok, d_in), jnp.float32),
        compiler_params=pltpu.CompilerParams(
            dimension_semantics=("parallel", "arbitrary")),
    )(acts, wdec_t_bf, bdec2)


def kernel(x, W_enc, b_enc, W_dec, b_dec):
    n_tok, d_in = x.shape
    d_sae = W_enc.shape[0]
    benc2 = b_enc.reshape(1, d_sae)
    bdec2 = b_dec.reshape(1, d_in)

    br1 = min(1024, n_tok)
    bs1 = min(1024, d_sae)
    pre, m8 = _encode(x, W_enc, benc2, bdec2, br1, bs1)

    br2 = min(128, n_tok)
    acts = _select(pre, m8, br2)

    wdec_bf = W_dec.astype(jnp.bfloat16)
    br3 = min(1024, n_tok)
    bk3 = min(2048, d_sae)
    recon = _decode(acts, wdec_bf, bdec2, br3, bk3)
    return (recon, acts)
